# Initial kernel scaffold; baseline (speedup 1.0000x reference)
import jax
import jax.numpy as jnp
from jax.experimental import pallas as pl


def kernel(x, edge_index, edge_attr, W1, b1, W2, b2, Wn, bn, gamma, beta):
    raise NotImplementedError("write your pallas kernel here")


# speedup vs baseline: 2.6488x; 2.6488x over previous
import functools

import jax
import jax.numpy as jnp
from jax import lax
from jax.experimental import pallas as pl
from jax.experimental.pallas import tpu as pltpu
from jax.experimental.pallas import tpu_sc as plsc

N = 10000
E = 320000
D = 128
ED = 16
H = 64

NC = 2
NS = 16
NW = NC * NS
EPW = E // NW
CH = 80
NCH = EPW // CH
ROWS_PER_TILE = N // NS

_f32 = jnp.float32


def _xw_body(x_ref, w_ref, o_ref):
    o_ref[...] = jnp.dot(x_ref[...], w_ref[...], preferred_element_type=_f32)


def _edge_body(g_ref, ea_ref, w1b_ref, b1_ref, w2_ref, b2_ref, o_ref):
    pre = g_ref[...] + jnp.dot(ea_ref[...], w1b_ref[...],
                               preferred_element_type=_f32) + b1_ref[...]
    h = jnp.maximum(pre, 0.0)
    o_ref[...] = jnp.maximum(
        jnp.dot(h, w2_ref[...], preferred_element_type=_f32) + b2_ref[...], 0.0)


def _node_body(x_ref, p0_ref, p1_ref, wna_ref, wnb_ref, bn_ref, gm_ref,
               bt_ref, o_ref):
    agg = p0_ref[...] + p1_ref[...]
    out = (jnp.dot(x_ref[...], wna_ref[...], preferred_element_type=_f32)
           + jnp.dot(agg, wnb_ref[...], preferred_element_type=_f32)
           + bn_ref[...])
    out = jnp.maximum(out, 0.0) + x_ref[...]
    mu = jnp.mean(out, axis=-1, keepdims=True)
    var = jnp.mean((out - mu) ** 2, axis=-1, keepdims=True)
    o_ref[...] = (out - mu) / jnp.sqrt(var + 1e-5) * gm_ref[...] + bt_ref[...]


def _gather_body(xw_hbm, j_hbm, g_hbm, idx_v, rows0, rows1, sem0, sem1):
    wid = lax.axis_index("s") * NC + lax.axis_index("c")
    base = wid * EPW
    pltpu.sync_copy(j_hbm.at[wid], idx_v)
    rows = (rows0, rows1)
    sems = (sem0, sem1)
    for b in range(2):
        pltpu.async_copy(xw_hbm.at[idx_v.at[b]], rows[b], sems[b])

    def body(it, carry):
        for b in range(2):
            ch = it * 2 + b

            @pl.when(ch < NCH)
            def _():
                pltpu.make_async_copy(xw_hbm.at[idx_v.at[ch]], rows[b],
                                      sems[b]).wait()
                pltpu.sync_copy(rows[b], g_hbm.at[pl.ds(base + ch * CH, CH)])

                @pl.when(ch + 2 < NCH)
                def _():
                    pltpu.async_copy(xw_hbm.at[idx_v.at[ch + 2]], rows[b],
                                     sems[b])
        return carry

    lax.fori_loop(0, (NCH + 1) // 2, body, 0)


def _scatter_body(eh_hbm, i_hbm, part_hbm, shared, idx_v, rows0, rows1, zbuf,
                  sem0, sem1):
    cid = lax.axis_index("c")
    sid = lax.axis_index("s")
    wid = sid * NC + cid
    base = wid * EPW
    pltpu.sync_copy(i_hbm.at[wid], idx_v)

    zero16 = jnp.zeros((16,), _f32)

    def zbody(r, carry):
        for q in range(H // 16):
            zbuf[r, pl.ds(q * 16, 16)] = zero16
        return carry

    lax.fori_loop(0, ROWS_PER_TILE, zbody, 0)
    pltpu.sync_copy(zbuf, shared.at[pl.ds(sid * ROWS_PER_TILE, ROWS_PER_TILE)])
    plsc.subcore_barrier()

    rows = (rows0, rows1)
    sems = (sem0, sem1)
    for b in range(2):
        pltpu.async_copy(eh_hbm.at[pl.ds(base + b * CH, CH)], rows[b], sems[b])

    def body(it, carry):
        for b in range(2):
            ch = it * 2 + b

            @pl.when(ch < NCH)
            def _():
                pltpu.make_async_copy(eh_hbm.at[pl.ds(base + ch * CH, CH)],
                                      rows[b], sems[b]).wait()
                pltpu.sync_copy(rows[b], shared.at[idx_v.at[ch]], add=True)

                @pl.when(ch + 2 < NCH)
                def _():
                    pltpu.async_copy(
                        eh_hbm.at[pl.ds(base + (ch + 2) * CH, CH)], rows[b],
                        sems[b])
        return carry

    lax.fori_loop(0, (NCH + 1) // 2, body, 0)
    plsc.subcore_barrier()

    pltpu.sync_copy(shared.at[pl.ds(sid * ROWS_PER_TILE, ROWS_PER_TILE)], zbuf)
    pltpu.sync_copy(
        zbuf, part_hbm.at[pl.ds(cid * N + sid * ROWS_PER_TILE, ROWS_PER_TILE)])


def kernel(x, edge_index, edge_attr, W1, b1, W2, b2, Wn, bn, gamma, beta):
    W1a = W1[:D]
    W1b = W1[D:]
    WnA = Wn[:D]
    WnB = Wn[D:]
    j3 = edge_index[1].reshape(NW, NCH, CH)
    i3 = edge_index[0].reshape(NW, NCH, CH)

    BN = 400
    GN = N // BN

    xw = pl.pallas_call(
        _xw_body,
        grid=(GN,),
        in_specs=[pl.BlockSpec((BN, D), lambda i: (i, 0)),
                  pl.BlockSpec((D, H), lambda i: (0, 0))],
        out_specs=pl.BlockSpec((BN, H), lambda i: (i, 0)),
        out_shape=jax.ShapeDtypeStruct((N, H), _f32),
    )(x, W1a)

    mesh = plsc.VectorSubcoreMesh(core_axis_name="c", subcore_axis_name="s")

    gather = pl.kernel(
        _gather_body,
        out_type=jax.ShapeDtypeStruct((E, H), _f32),
        mesh=mesh,
        compiler_params=pltpu.CompilerParams(use_tc_tiling_on_sc=False),
        scratch_types=[
            pltpu.VMEM((NCH, CH), jnp.int32),
            pltpu.VMEM((CH, H), _f32),
            pltpu.VMEM((CH, H), _f32),
            pltpu.SemaphoreType.DMA,
            pltpu.SemaphoreType.DMA,
        ],
    )
    g = gather(xw, j3)

    BE = 2000
    GE = E // BE
    eh = pl.pallas_call(
        _edge_body,
        grid=(GE,),
        in_specs=[pl.BlockSpec((BE, H), lambda i: (i, 0)),
                  pl.BlockSpec((BE, ED), lambda i: (i, 0)),
                  pl.BlockSpec((ED, H), lambda i: (0, 0)),
                  pl.BlockSpec((1, H), lambda i: (0, 0)),
                  pl.BlockSpec((H, H), lambda i: (0, 0)),
                  pl.BlockSpec((1, H), lambda i: (0, 0))],
        out_specs=pl.BlockSpec((BE, H), lambda i: (i, 0)),
        out_shape=jax.ShapeDtypeStruct((E, H), _f32),
    )(g, edge_attr, W1b, b1.reshape(1, H), W2, b2.reshape(1, H))

    scatter = pl.kernel(
        _scatter_body,
        out_type=jax.ShapeDtypeStruct((NC * N, H), _f32),
        mesh=mesh,
        compiler_params=pltpu.CompilerParams(use_tc_tiling_on_sc=False),
        scratch_types=[
            pltpu.VMEM_SHARED((N, H), _f32),
            pltpu.VMEM((NCH, CH), jnp.int32),
            pltpu.VMEM((CH, H), _f32),
            pltpu.VMEM((CH, H), _f32),
            pltpu.VMEM((ROWS_PER_TILE, H), _f32),
            pltpu.SemaphoreType.DMA,
            pltpu.SemaphoreType.DMA,
        ],
    )
    parts = scatter(eh, i3)

    out = pl.pallas_call(
        _node_body,
        grid=(GN,),
        in_specs=[pl.BlockSpec((BN, D), lambda i: (i, 0)),
                  pl.BlockSpec((BN, H), lambda i: (i, 0)),
                  pl.BlockSpec((BN, H), lambda i: (i + GN, 0)),
                  pl.BlockSpec((D, D), lambda i: (0, 0)),
                  pl.BlockSpec((H, D), lambda i: (0, 0)),
                  pl.BlockSpec((1, D), lambda i: (0, 0)),
                  pl.BlockSpec((1, D), lambda i: (0, 0)),
                  pl.BlockSpec((1, D), lambda i: (0, 0))],
        out_specs=pl.BlockSpec((BN, D), lambda i: (i, 0)),
        out_shape=jax.ShapeDtypeStruct((N, D), _f32),
    )(x, parts, parts, WnA, WnB, bn.reshape(1, D), gamma.reshape(1, D),
      beta.reshape(1, D))
    return out


# speedup vs baseline: 4.5287x; 1.7097x over previous
import jax
import jax.numpy as jnp
from jax import lax
from jax.experimental import pallas as pl
from jax.experimental.pallas import tpu as pltpu
from jax.experimental.pallas import tpu_sc as plsc

N = 10000
E = 320000
D = 128
ED = 16
H = 64

NC = 2
NS = 16
NW = NC * NS
EPW = E // NW
CH = 80
NCH = EPW // CH
ROWS_PER_TILE = N // NS

_f32 = jnp.float32


def _blockdiag(w):
    z = jnp.zeros_like(w)
    return jnp.concatenate(
        [jnp.concatenate([w, z], axis=1), jnp.concatenate([z, w], axis=1)],
        axis=0)


def _xw_body(x2_ref, w_ref, o_ref):
    o_ref[...] = jnp.dot(x2_ref[...], w_ref[...], preferred_element_type=_f32)


def _edge_body(g2_ref, ea2_ref, w1b2_ref, b12_ref, w2d_ref, b22_ref, o_ref):
    pre = g2_ref[...] + jnp.dot(ea2_ref[...], w1b2_ref[...],
                                preferred_element_type=_f32) + b12_ref[...]
    h = jnp.maximum(pre, 0.0)
    o_ref[...] = jnp.maximum(
        jnp.dot(h, w2d_ref[...], preferred_element_type=_f32) + b22_ref[...],
        0.0)


def _node_body(x2_ref, v0_ref, v1_ref, wna2_ref, wnb2_ref, bn2_ref, gm2_ref,
               bt2_ref, o_ref):
    agg2 = v0_ref[...] + v1_ref[...]
    out = (jnp.dot(x2_ref[...], wna2_ref[...], preferred_element_type=_f32)
           + jnp.dot(agg2, wnb2_ref[...], preferred_element_type=_f32)
           + bn2_ref[...])
    out = jnp.maximum(out, 0.0) + x2_ref[...]
    o_l = out[:, :D]
    o_r = out[:, D:]
    mu_l = jnp.mean(o_l, axis=-1, keepdims=True)
    mu_r = jnp.mean(o_r, axis=-1, keepdims=True)
    var_l = jnp.mean((o_l - mu_l) ** 2, axis=-1, keepdims=True)
    var_r = jnp.mean((o_r - mu_r) ** 2, axis=-1, keepdims=True)
    n_l = (o_l - mu_l) / jnp.sqrt(var_l + 1e-5)
    n_r = (o_r - mu_r) / jnp.sqrt(var_r + 1e-5)
    nrm = jnp.concatenate([n_l, n_r], axis=1)
    o_ref[...] = nrm * gm2_ref[...] + bt2_ref[...]


def _gather_body(xw_hbm, j_hbm, g_hbm, idx_v, rows0, rows1, sem0, sem1):
    wid = lax.axis_index("s") * NC + lax.axis_index("c")
    base = wid * EPW
    pltpu.sync_copy(j_hbm.at[wid], idx_v)
    rows = (rows0, rows1)
    sems = (sem0, sem1)
    for b in range(2):
        pltpu.async_copy(xw_hbm.at[idx_v.at[b]], rows[b], sems[b])

    def body(it, carry):
        for b in range(2):
            ch = it * 2 + b

            @pl.when(ch < NCH)
            def _():
                pltpu.make_async_copy(xw_hbm.at[idx_v.at[ch]], rows[b],
                                      sems[b]).wait()
                pltpu.sync_copy(rows[b], g_hbm.at[pl.ds(base + ch * CH, CH)])

                @pl.when(ch + 2 < NCH)
                def _():
                    pltpu.async_copy(xw_hbm.at[idx_v.at[ch + 2]], rows[b],
                                     sems[b])
        return carry

    lax.fori_loop(0, (NCH + 1) // 2, body, 0)


def _scatter_body(eh_hbm, i_hbm, part_hbm, shared, idx_v, rows0, rows1, zbuf,
                  sem0, sem1):
    cid = lax.axis_index("c")
    sid = lax.axis_index("s")
    wid = sid * NC + cid
    base = wid * EPW
    pltpu.sync_copy(i_hbm.at[wid], idx_v)

    zero16 = jnp.zeros((16,), _f32)

    def zbody(r, carry):
        for q in range(H // 16):
            zbuf[r, pl.ds(q * 16, 16)] = zero16
        return carry

    lax.fori_loop(0, ROWS_PER_TILE, zbody, 0)
    pltpu.sync_copy(zbuf, shared.at[pl.ds(sid * ROWS_PER_TILE, ROWS_PER_TILE)])
    plsc.subcore_barrier()

    rows = (rows0, rows1)
    sems = (sem0, sem1)
    for b in range(2):
        pltpu.async_copy(eh_hbm.at[pl.ds(base + b * CH, CH)], rows[b], sems[b])

    def body(it, carry):
        for b in range(2):
            ch = it * 2 + b

            @pl.when(ch < NCH)
            def _():
                pltpu.make_async_copy(eh_hbm.at[pl.ds(base + ch * CH, CH)],
                                      rows[b], sems[b]).wait()
                pltpu.sync_copy(rows[b], shared.at[idx_v.at[ch]], add=True)

                @pl.when(ch + 2 < NCH)
                def _():
                    pltpu.async_copy(
                        eh_hbm.at[pl.ds(base + (ch + 2) * CH, CH)], rows[b],
                        sems[b])
        return carry

    lax.fori_loop(0, (NCH + 1) // 2, body, 0)
    plsc.subcore_barrier()

    pltpu.sync_copy(shared.at[pl.ds(sid * ROWS_PER_TILE, ROWS_PER_TILE)], zbuf)
    pltpu.sync_copy(
        zbuf, part_hbm.at[pl.ds(cid * N + sid * ROWS_PER_TILE, ROWS_PER_TILE)])


def kernel(x, edge_index, edge_attr, W1, b1, W2, b2, Wn, bn, gamma, beta):
    W1a2 = _blockdiag(W1[:D])
    W1b2 = _blockdiag(W1[D:])
    W2d = _blockdiag(W2)
    WnA2 = _blockdiag(Wn[:D])
    WnB2 = _blockdiag(Wn[D:])
    b12 = jnp.tile(b1, 2).reshape(1, 2 * H)
    b22 = jnp.tile(b2, 2).reshape(1, 2 * H)
    bn2 = jnp.tile(bn, 2).reshape(1, 2 * D)
    gm2 = jnp.tile(gamma, 2).reshape(1, 2 * D)
    bt2 = jnp.tile(beta, 2).reshape(1, 2 * D)
    j3 = edge_index[1].reshape(NW, NCH, CH)
    i3 = edge_index[0].reshape(NW, NCH, CH)
    x2 = x.reshape(N // 2, 2 * D)
    ea2 = edge_attr.reshape(E // 2, 2 * ED)

    BN2 = 200
    GN2 = (N // 2) // BN2

    xw2 = pl.pallas_call(
        _xw_body,
        grid=(GN2,),
        in_specs=[pl.BlockSpec((BN2, 2 * D), lambda i: (i, 0)),
                  pl.BlockSpec((2 * D, 2 * H), lambda i: (0, 0))],
        out_specs=pl.BlockSpec((BN2, 2 * H), lambda i: (i, 0)),
        out_shape=jax.ShapeDtypeStruct((N // 2, 2 * H), _f32),
    )(x2, W1a2)
    xw = xw2.reshape(N, H)

    mesh = plsc.VectorSubcoreMesh(core_axis_name="c", subcore_axis_name="s")

    gather = pl.kernel(
        _gather_body,
        out_type=jax.ShapeDtypeStruct((E, H), _f32),
        mesh=mesh,
        compiler_params=pltpu.CompilerParams(use_tc_tiling_on_sc=False),
        scratch_types=[
            pltpu.VMEM((NCH, CH), jnp.int32),
            pltpu.VMEM((CH, H), _f32),
            pltpu.VMEM((CH, H), _f32),
            pltpu.SemaphoreType.DMA,
            pltpu.SemaphoreType.DMA,
        ],
    )
    g2 = gather(xw, j3).reshape(E // 2, 2 * H)

    BE2 = 2000
    GE2 = (E // 2) // BE2
    eh2 = pl.pallas_call(
        _edge_body,
        grid=(GE2,),
        in_specs=[pl.BlockSpec((BE2, 2 * H), lambda i: (i, 0)),
                  pl.BlockSpec((BE2, 2 * ED), lambda i: (i, 0)),
                  pl.BlockSpec((2 * ED, 2 * H), lambda i: (0, 0)),
                  pl.BlockSpec((1, 2 * H), lambda i: (0, 0)),
                  pl.BlockSpec((2 * H, 2 * H), lambda i: (0, 0)),
                  pl.BlockSpec((1, 2 * H), lambda i: (0, 0))],
        out_specs=pl.BlockSpec((BE2, 2 * H), lambda i: (i, 0)),
        out_shape=jax.ShapeDtypeStruct((E // 2, 2 * H), _f32),
    )(g2, ea2, W1b2, b12, W2d, b22)
    eh = eh2.reshape(E, H)

    scatter = pl.kernel(
        _scatter_body,
        out_type=jax.ShapeDtypeStruct((NC * N, H), _f32),
        mesh=mesh,
        compiler_params=pltpu.CompilerParams(use_tc_tiling_on_sc=False),
        scratch_types=[
            pltpu.VMEM_SHARED((N, H), _f32),
            pltpu.VMEM((NCH, CH), jnp.int32),
            pltpu.VMEM((CH, H), _f32),
            pltpu.VMEM((CH, H), _f32),
            pltpu.VMEM((ROWS_PER_TILE, H), _f32),
            pltpu.SemaphoreType.DMA,
            pltpu.SemaphoreType.DMA,
        ],
    )
    parts = scatter(eh, i3)
    v = parts.reshape(N, 2 * H)

    out2 = pl.pallas_call(
        _node_body,
        grid=(GN2,),
        in_specs=[pl.BlockSpec((BN2, 2 * D), lambda i: (i, 0)),
                  pl.BlockSpec((BN2, 2 * H), lambda i: (i, 0)),
                  pl.BlockSpec((BN2, 2 * H), lambda i: (i + GN2, 0)),
                  pl.BlockSpec((2 * D, 2 * D), lambda i: (0, 0)),
                  pl.BlockSpec((2 * H, 2 * D), lambda i: (0, 0)),
                  pl.BlockSpec((1, 2 * D), lambda i: (0, 0)),
                  pl.BlockSpec((1, 2 * D), lambda i: (0, 0)),
                  pl.BlockSpec((1, 2 * D), lambda i: (0, 0))],
        out_specs=pl.BlockSpec((BN2, 2 * D), lambda i: (i, 0)),
        out_shape=jax.ShapeDtypeStruct((N // 2, 2 * D), _f32),
    )(x2, v, v, WnA2, WnB2, bn2, gm2, bt2)
    return out2.reshape(N, D)
